# Initial kernel scaffold; baseline (speedup 1.0000x reference)
import jax
import jax.numpy as jnp
from jax.experimental import pallas as pl


def kernel(x, edge_index, batch, params):
    raise NotImplementedError("write your pallas kernel here")


# speedup vs baseline: 2.4488x; 2.4488x over previous
import functools

import jax
import jax.numpy as jnp
from jax import lax
from jax.experimental import pallas as pl
from jax.experimental.pallas import tpu as pltpu
from jax.experimental.pallas import tpu_sc as plsc

_N = 100000
_E = 1600000
_C = 32
_G = 256
_V = 128
_BLK = 2048
_NBLK = -(-_N // _BLK)
_PBLK = _BLK * 16 // 128
_PROWS = 12504
_NPAD = _PROWS * 128 // 16
_R = 12500
_NC = 2
_NS = 16
_SPAN = 34816
_NCALL = 3
_HBLK = _SPAN // _BLK
_STRIPE = 2304
_SROWS = _STRIPE * _NS
_DUMP = 36000
_ZROWS = 288
_PSTRIPE = _STRIPE * 16 // 128
_AROWS = _SROWS * 16 // 128


def _sc_mesh():
    return plsc.VectorSubcoreMesh(
        core_axis_name="c", subcore_axis_name="s",
        num_cores=_NC, num_subcores=_NS)


def _redirect(idst, j, base):
    row = idst.at[j]
    for z in range(8):
        v = row[pl.ds(z * 16, 16)]
        t = v - base
        ok = (t >= 0) & (t < _SPAN)
        row[pl.ds(z * 16, 16)] = jnp.where(ok, t, _DUMP)


def _dump_packed(acc_sh, zbuf, packed, out_hbm, c, s):
    def rep_chunk(ch, carry):
        pltpu.sync_copy(
            acc_sh.at[pl.ds(s * _STRIPE + ch * _ZROWS, _ZROWS)], zbuf)

        def rows_body(q, carry2):
            for kk in range(8):
                packed[ch * 36 + q, pl.ds(kk * 16, 16)] = zbuf[q * 8 + kk]
            return carry2

        lax.fori_loop(0, 36, rows_body, 0)
        return carry

    lax.fori_loop(0, 8, rep_chunk, 0)
    pltpu.sync_copy(packed, out_hbm.at[c, s])


def _zero_acc(z_hbm, zbuf, acc_sh, s):
    pltpu.sync_copy(z_hbm, zbuf)
    for z in range(8):
        pltpu.sync_copy(zbuf, acc_sh.at[pl.ds(s * _STRIPE + z * _ZROWS, _ZROWS)])


import numpy as _np


def _edge_rowtab():
    tab = _np.full((784, 128), _R - 1, _np.int32)
    for s in range(16):
        tbase = s * 781 + min(s, 4)
        for m in range(49):
            rows = _np.minimum(tbase + m * 16 + _np.arange(16), _R - 1)
            tab[s * 49 + m, :16] = rows
    return jnp.asarray(tab)


def _sc_edge_scatter(src2, dst2, hta, htb, zeros, half):
    base = half * _SPAN

    @functools.partial(
        pl.kernel,
        out_type=jax.ShapeDtypeStruct((2, _NS, _PSTRIPE, 128), jnp.float32),
        mesh=_sc_mesh(),
        compiler_params=pltpu.CompilerParams(use_tc_tiling_on_sc=False),
        scratch_types=[
            pltpu.VMEM_SHARED((_SROWS, 16), jnp.float32),
            pltpu.VMEM((128,), jnp.int32),
            pltpu.VMEM((16, 128), jnp.int32),
            pltpu.VMEM((16, 128), jnp.int32),
            pltpu.VMEM((128, 16), jnp.float32),
            pltpu.VMEM((_ZROWS, 16), jnp.float32),
            pltpu.VMEM((_PSTRIPE, 128), jnp.float32),
            pltpu.SemaphoreType.DMA,
        ],
    )
    def k(rowtab_hbm, src_hbm, dst_hbm, hta_hbm, htb_hbm, z_hbm, acc_hbm,
          acc_sh, rowbuf, isrc, idst, rows, zbuf, packed, sem):
        c = lax.axis_index("c")
        s = lax.axis_index("s")
        _zero_acc(z_hbm, zbuf, acc_sh, s)
        plsc.subcore_barrier()

        n_rows = 781 + jnp.where(s < 4, 1, 0)

        def do_chunk(ch, valid=None):
            pltpu.sync_copy(rowtab_hbm.at[ch], rowbuf)
            irow = rowbuf.at[pl.ds(0, 16)]
            pltpu.async_copy(src_hbm.at[irow], isrc, sem).wait()
            pltpu.async_copy(dst_hbm.at[irow], idst, sem).wait()
            for j in range(16):
                def scatter_one(j=j):
                    _redirect(idst, j, base)

                    @pl.when(c == 0)
                    def _():
                        pltpu.async_copy(
                            hta_hbm.at[isrc.at[j]], rows, sem).wait()

                    @pl.when(c == 1)
                    def _():
                        pltpu.async_copy(
                            htb_hbm.at[isrc.at[j]], rows, sem).wait()

                    pltpu.sync_copy(rows, acc_sh.at[idst.at[j]], add=True)
                if valid is None:
                    scatter_one()
                else:
                    pl.when(j < valid)(scatter_one)

        def macro(m, carry):
            do_chunk(s * 49 + m)
            return carry

        lax.fori_loop(0, 48, macro, 0)
        do_chunk(s * 49 + 48, n_rows - 768)
        plsc.subcore_barrier()
        _dump_packed(acc_sh, zbuf, packed, acc_hbm, c, s)

    return k(_edge_rowtab(), src2, dst2, hta, htb, zeros)


def _sc_degree(dst2, ones, zeros, half):
    base = half * _SPAN

    @functools.partial(
        pl.kernel,
        out_type=jax.ShapeDtypeStruct((2, _NS, _PSTRIPE, 128), jnp.float32),
        mesh=_sc_mesh(),
        compiler_params=pltpu.CompilerParams(use_tc_tiling_on_sc=False),
        scratch_types=[
            pltpu.VMEM_SHARED((_SROWS, 16), jnp.float32),
            pltpu.VMEM((16,), jnp.int32),
            pltpu.VMEM((16, 128), jnp.int32),
            pltpu.VMEM((128, 16), jnp.float32),
            pltpu.VMEM((_ZROWS, 16), jnp.float32),
            pltpu.VMEM((_PSTRIPE, 128), jnp.float32),
            pltpu.SemaphoreType.DMA,
        ],
    )
    def k(dst_hbm, ones_hbm, z_hbm, deg_hbm,
          acc_sh, irow, idst, ones_v, zbuf, packed, sem):
        c = lax.axis_index("c")
        s = lax.axis_index("s")
        _zero_acc(z_hbm, zbuf, acc_sh, s)
        pltpu.sync_copy(ones_hbm, ones_v)
        plsc.subcore_barrier()
        w = s * _NC + c
        tbase = w * 390 + jnp.minimum(w, 20)
        n_rows = 390 + jnp.where(w < 20, 1, 0)

        def do_chunk(row0, valid=None):
            irow[...] = jnp.minimum(row0 + lax.iota(jnp.int32, 16), _R - 1)
            pltpu.async_copy(dst_hbm.at[irow], idst, sem).wait()
            for j in range(16):
                def scatter_one(j=j):
                    _redirect(idst, j, base)
                    pltpu.sync_copy(ones_v, acc_sh.at[idst.at[j]], add=True)
                if valid is None:
                    scatter_one()
                else:
                    pl.when(j < valid)(scatter_one)

        def macro(m, carry):
            do_chunk(tbase + m * 16)
            return carry

        lax.fori_loop(0, 24, macro, 0)
        do_chunk(tbase + 384, n_rows - 384)
        plsc.subcore_barrier()
        _dump_packed(acc_sh, zbuf, packed, deg_hbm, c, s)

    return k(dst2, ones, zeros)


def _unpack128(a):
    cols = [a[:, k * 16:(k + 1) * 16] for k in range(8)]
    return jnp.reshape(jnp.stack(cols, axis=1), (_BLK, 16))


def _full(shape):
    return pl.BlockSpec(shape, lambda i: tuple(0 for _ in shape))


def _acc_specs():
    def span_map(t, ch):
        return lambda i: (ch, jnp.clip(i - t * _HBLK, 0, _HBLK - 1), 0)
    specs = []
    for t in range(_NCALL):
        specs.append(pl.BlockSpec((1, _PBLK, 128), span_map(t, 0)))
        specs.append(pl.BlockSpec((1, _PBLK, 128), span_map(t, 1)))
    return specs


def _sel_unpack(i, refs):
    acc_a = _unpack128(refs[0][0])
    acc_b = _unpack128(refs[1][0])
    for t in range(1, _NCALL):
        in_t = i >= t * _HBLK
        acc_a = jnp.where(in_t, _unpack128(refs[2 * t][0]), acc_a)
        acc_b = jnp.where(in_t, _unpack128(refs[2 * t + 1][0]), acc_b)
    return jnp.concatenate([acc_a, acc_b], axis=1)


def _tc_prologue(x2, degs, emb, w1, b1, w2, b2, wc0):
    def body(x_ref, dg0a, dg0b, dg1a, dg1b, dg2a, dg2b, emb_ref,
             w1r, b1r, w2r, b2r,
             wc0r, h0_ref, hta_ref, htb_ref, dis_ref):
        i = pl.program_id(0)
        t = jnp.maximum(jnp.dot(emb_ref[...], w1r[...],
                                preferred_element_type=jnp.float32) + b1r[...], 0.0)
        t = jnp.maximum(jnp.dot(t, w2r[...],
                                preferred_element_type=jnp.float32) + b2r[...], 0.0)
        x = x_ref[:, 0]
        oh = (x[:, None] == lax.broadcasted_iota(jnp.int32, (_BLK, _V), 1)
              ).astype(jnp.float32)
        h0 = jnp.dot(oh, t, preferred_element_type=jnp.float32)
        du = _sel_unpack(i, [dg0a, dg0b, dg1a, dg1b, dg2a, dg2b])
        deg = du[:, 0:1] + du[:, 16:17] + 1.0
        dis = lax.rsqrt(deg)
        h0_ref[...] = h0
        dis_ref[...] = dis
        ht = jnp.dot(h0, wc0r[...], preferred_element_type=jnp.float32) * dis
        ht3 = jnp.reshape(ht, (_PBLK, 8, 32))
        for k in range(8):
            hta_ref[:, k * 16:(k + 1) * 16] = ht3[:, k, :16]
            htb_ref[:, k * 16:(k + 1) * 16] = ht3[:, k, 16:]

    return pl.pallas_call(
        body,
        grid=(_NBLK,),
        in_specs=[pl.BlockSpec((_BLK, 1), lambda i: (i, 0))] + _acc_specs() + [
            _full((_V, _C)), _full((_C, _C)), _full((1, _C)),
            _full((_C, _C)), _full((1, _C)), _full((_C, _C)),
        ],
        out_specs=[
            pl.BlockSpec((_BLK, _C), lambda i: (i, 0)),
            pl.BlockSpec((_PBLK, 128), lambda i: (i, 0)),
            pl.BlockSpec((_PBLK, 128), lambda i: (i, 0)),
            pl.BlockSpec((_BLK, 1), lambda i: (i, 0)),
        ],
        out_shape=[
            jax.ShapeDtypeStruct((_N, _C), jnp.float32),
            jax.ShapeDtypeStruct((_PROWS, 128), jnp.float32),
            jax.ShapeDtypeStruct((_PROWS, 128), jnp.float32),
            jax.ShapeDtypeStruct((_N, 1), jnp.float32),
        ],
    )(x2, degs[0], degs[0], degs[1], degs[1], degs[2], degs[2],
      emb, w1, b1, w2, b2, wc0)


def _tc_conv(h, accs, dis, wl, bl, wn):
    last = wn is None

    def body(h_ref, a0, b0_, a1, b1_, a2, b2_, dis_ref, wlr, blr, *rest):
        if last:
            (h_out,) = rest
        else:
            wnr, h_out, hta_out, htb_out = rest
        i = pl.program_id(0)
        dis = dis_ref[...]
        hw = jnp.dot(h_ref[...], wlr[...], preferred_element_type=jnp.float32)
        acc = _sel_unpack(i, [a0, b0_, a1, b1_, a2, b2_])
        out = jnp.maximum(dis * acc + (dis * dis) * hw + blr[...], 0.0)
        h_out[...] = out
        if not last:
            ht = jnp.dot(out, wnr[...], preferred_element_type=jnp.float32) * dis
            ht3 = jnp.reshape(ht, (_PBLK, 8, 32))
            for k in range(8):
                hta_out[:, k * 16:(k + 1) * 16] = ht3[:, k, :16]
                htb_out[:, k * 16:(k + 1) * 16] = ht3[:, k, 16:]

    in_specs = [pl.BlockSpec((_BLK, _C), lambda i: (i, 0))] + _acc_specs() + [
        pl.BlockSpec((_BLK, 1), lambda i: (i, 0)),
        _full((_C, _C)), _full((1, _C)),
    ]
    out_specs = [pl.BlockSpec((_BLK, _C), lambda i: (i, 0))]
    out_shape = [jax.ShapeDtypeStruct((_N, _C), jnp.float32)]
    args = [h, accs[0], accs[0], accs[1], accs[1], accs[2], accs[2],
            dis, wl, bl]
    if not last:
        in_specs.append(_full((_C, _C)))
        out_specs.append(pl.BlockSpec((_PBLK, 128), lambda i: (i, 0)))
        out_specs.append(pl.BlockSpec((_PBLK, 128), lambda i: (i, 0)))
        out_shape.append(jax.ShapeDtypeStruct((_PROWS, 128), jnp.float32))
        out_shape.append(jax.ShapeDtypeStruct((_PROWS, 128), jnp.float32))
        args.append(wn)

    res = pl.pallas_call(
        body, grid=(_NBLK,), in_specs=in_specs,
        out_specs=out_specs, out_shape=out_shape,
    )(*args)
    return (res[0], None, None) if last else (res[0], res[1], res[2])


def _tc_pool(batch2, h):
    def body(b_ref, h_ref, g_ref):
        i = pl.program_id(0)

        @pl.when(i == 0)
        def _():
            g_ref[...] = jnp.zeros_like(g_ref)

        b = b_ref[:, 0]
        row = i * _BLK + lax.broadcasted_iota(jnp.int32, (_BLK, _G), 0)
        oh = ((b[:, None] == lax.broadcasted_iota(jnp.int32, (_BLK, _G), 1))
              & (row < _N)).astype(jnp.float32)
        g_ref[...] += lax.dot_general(
            oh, h_ref[...], (((0,), (0,)), ((), ())),
            preferred_element_type=jnp.float32)

    return pl.pallas_call(
        body,
        grid=(_NBLK,),
        in_specs=[
            pl.BlockSpec((_BLK, 1), lambda i: (i, 0)),
            pl.BlockSpec((_BLK, _C), lambda i: (i, 0)),
        ],
        out_specs=pl.BlockSpec((_G, _C), lambda i: (0, 0)),
        out_shape=jax.ShapeDtypeStruct((_G, _C), jnp.float32),
    )(batch2, h)


def _tc_head(g, w1, b1, w2, b2, wp, bp):
    def body(g_ref, w1r, b1r, w2r, b2r, wpr, bpr, o_ref):
        t = jnp.maximum(jnp.dot(g_ref[...], w1r[...],
                                preferred_element_type=jnp.float32) + b1r[...], 0.0)
        t = jnp.maximum(jnp.dot(t, w2r[...],
                                preferred_element_type=jnp.float32) + b2r[...], 0.0)
        o_ref[...] = jnp.dot(t, wpr[...],
                             preferred_element_type=jnp.float32) + bpr[...]

    f0 = lambda shape: pl.BlockSpec(shape, lambda: tuple(0 for _ in shape))
    return pl.pallas_call(
        body,
        in_specs=[f0((_G, _C)), f0((_C, _C)), f0((1, _C)),
                  f0((_C, _C)), f0((1, _C)), f0((_C, 1)),
                  f0((1, 1))],
        out_specs=f0((_G, 1)),
        out_shape=jax.ShapeDtypeStruct((_G, 1), jnp.float32),
    )(g, w1, b1, w2, b2, wp, bp)


def kernel(x, edge_index, batch, params):
    src2 = edge_index[0].astype(jnp.int32).reshape(_R, 128)
    dst2 = edge_index[1].astype(jnp.int32).reshape(_R, 128)
    zeros = jnp.zeros((_ZROWS, 16), jnp.float32)
    ones = jnp.ones((128, 16), jnp.float32)

    x2 = x.astype(jnp.int32).reshape(_N, 1)
    batch2 = batch.astype(jnp.int32).reshape(_N, 1)

    p = params
    b = lambda v: v.reshape(1, -1)

    degs = [_sc_degree(dst2, ones, zeros, t).reshape(2, _AROWS, 128)
            for t in range(_NCALL)]

    h, hta, htb, dis = _tc_prologue(
        x2, degs, p["emb"],
        p["pre"][0]["W"], b(p["pre"][0]["b"]),
        p["pre"][1]["W"], b(p["pre"][1]["b"]),
        p["convs"][0]["W"])

    n_convs = len(p["convs"])
    for l in range(n_convs):
        ha = hta.reshape(_NPAD, 16)
        hb = htb.reshape(_NPAD, 16)
        accs = [_sc_edge_scatter(src2, dst2, ha, hb, zeros, t
                                 ).reshape(2, _AROWS, 128)
                for t in range(_NCALL)]
        wn = p["convs"][l + 1]["W"] if l + 1 < n_convs else None
        h, hta, htb = _tc_conv(h, accs, dis, p["convs"][l]["W"],
                               b(p["convs"][l]["b"]), wn)

    g = _tc_pool(batch2, h)
    return _tc_head(
        g,
        p["post"][0]["W"], b(p["post"][0]["b"]),
        p["post"][1]["W"], b(p["post"][1]["b"]),
        p["prop"]["W"], b(p["prop"]["b"].reshape(1, 1)))


# speedup vs baseline: 2.4522x; 1.0014x over previous
import functools

import jax
import jax.numpy as jnp
from jax import lax
from jax.experimental import pallas as pl
from jax.experimental.pallas import tpu as pltpu
from jax.experimental.pallas import tpu_sc as plsc

_N = 100000
_E = 1600000
_C = 32
_G = 256
_V = 128
_BLK = 2048
_NBLK = -(-_N // _BLK)
_PBLK = _BLK * 16 // 128
_PROWS = 12504
_NPAD = _PROWS * 128 // 16
_R = 12500
_NC = 2
_NS = 16
_SPAN = 34816
_NCALL = 3
_HBLK = _SPAN // _BLK
_STRIPE = 2304
_SROWS = _STRIPE * _NS
_DUMP = 36000
_ZROWS = 288
_PSTRIPE = _STRIPE * 16 // 128
_AROWS = _SROWS * 16 // 128


def _sc_mesh():
    return plsc.VectorSubcoreMesh(
        core_axis_name="c", subcore_axis_name="s",
        num_cores=_NC, num_subcores=_NS)


def _redirect(idst, j, base):
    row = idst.at[j]
    for z in range(8):
        v = row[pl.ds(z * 16, 16)]
        t = v - base
        ok = (t >= 0) & (t < _SPAN)
        row[pl.ds(z * 16, 16)] = jnp.where(ok, t, _DUMP)


def _dump_packed(acc_sh, zbuf, packed, out_hbm, c, s):
    def rep_chunk(ch, carry):
        pltpu.sync_copy(
            acc_sh.at[pl.ds(s * _STRIPE + ch * _ZROWS, _ZROWS)], zbuf)

        def rows_body(q, carry2):
            for kk in range(8):
                packed[ch * 36 + q, pl.ds(kk * 16, 16)] = zbuf[q * 8 + kk]
            return carry2

        lax.fori_loop(0, 36, rows_body, 0)
        return carry

    lax.fori_loop(0, 8, rep_chunk, 0)
    pltpu.sync_copy(packed, out_hbm.at[c, s])


def _zero_acc(z_hbm, zbuf, acc_sh, s):
    pltpu.sync_copy(z_hbm, zbuf)
    for z in range(8):
        pltpu.sync_copy(zbuf, acc_sh.at[pl.ds(s * _STRIPE + z * _ZROWS, _ZROWS)])


import numpy as _np


def _edge_rowtab():
    tab = _np.full((784, 128), _R - 1, _np.int32)
    for s in range(16):
        tbase = s * 781 + min(s, 4)
        for m in range(49):
            rows = _np.minimum(tbase + m * 16 + _np.arange(16), _R - 1)
            tab[s * 49 + m, :16] = rows
    return jnp.asarray(tab)


def _sc_edge_scatter(src2, dst2, hta, htb, zeros, half):
    base = half * _SPAN

    @functools.partial(
        pl.kernel,
        out_type=jax.ShapeDtypeStruct((2, _NS, _PSTRIPE, 128), jnp.float32),
        mesh=_sc_mesh(),
        compiler_params=pltpu.CompilerParams(use_tc_tiling_on_sc=False),
        scratch_types=[
            pltpu.VMEM_SHARED((_SROWS, 16), jnp.float32),
            pltpu.VMEM((128,), jnp.int32),
            pltpu.VMEM((16, 128), jnp.int32),
            pltpu.VMEM((16, 128), jnp.int32),
            pltpu.VMEM((128, 16), jnp.float32),
            pltpu.VMEM((128, 16), jnp.float32),
            pltpu.VMEM((_ZROWS, 16), jnp.float32),
            pltpu.VMEM((_PSTRIPE, 128), jnp.float32),
            pltpu.SemaphoreType.DMA,
        ],
    )
    def k(rowtab_hbm, src_hbm, dst_hbm, hta_hbm, htb_hbm, z_hbm, acc_hbm,
          acc_sh, rowbuf, isrc, idst, rows0, rows1, zbuf, packed, sem):
        c = lax.axis_index("c")
        s = lax.axis_index("s")
        _zero_acc(z_hbm, zbuf, acc_sh, s)
        plsc.subcore_barrier()

        n_rows = 781 + jnp.where(s < 4, 1, 0)

        bufs = (rows0, rows1)

        def do_chunk(ch, valid=None):
            pltpu.sync_copy(rowtab_hbm.at[ch], rowbuf)
            irow = rowbuf.at[pl.ds(0, 16)]
            pltpu.async_copy(src_hbm.at[irow], isrc, sem).wait()
            pltpu.async_copy(dst_hbm.at[irow], idst, sem).wait()
            if valid is None:
                for j in range(16):
                    _redirect(idst, j, base)

                def sched(src_ref, j):
                    if j == 0:
                        pltpu.async_copy(
                            src_ref.at[isrc.at[0]], bufs[0], sem)
                    pltpu.make_async_copy(
                        src_ref.at[isrc.at[j]], bufs[j % 2], sem).wait()
                    if j + 1 < 16:
                        pltpu.async_copy(
                            src_ref.at[isrc.at[j + 1]], bufs[(j + 1) % 2], sem)

                for j in range(16):
                    pl.when(c == 0)(functools.partial(sched, hta_hbm, j))
                    pl.when(c == 1)(functools.partial(sched, htb_hbm, j))
                    pltpu.sync_copy(bufs[j % 2], acc_sh.at[idst.at[j]],
                                    add=True)
            else:
                for j in range(16):
                    def scatter_one(j=j):
                        _redirect(idst, j, base)

                        @pl.when(c == 0)
                        def _():
                            pltpu.async_copy(
                                hta_hbm.at[isrc.at[j]], rows0, sem).wait()

                        @pl.when(c == 1)
                        def _():
                            pltpu.async_copy(
                                htb_hbm.at[isrc.at[j]], rows0, sem).wait()

                        pltpu.sync_copy(rows0, acc_sh.at[idst.at[j]],
                                        add=True)
                    pl.when(j < valid)(scatter_one)

        def macro(m, carry):
            do_chunk(s * 49 + m)
            return carry

        lax.fori_loop(0, 48, macro, 0)
        do_chunk(s * 49 + 48, n_rows - 768)
        plsc.subcore_barrier()
        _dump_packed(acc_sh, zbuf, packed, acc_hbm, c, s)

    return k(_edge_rowtab(), src2, dst2, hta, htb, zeros)


def _sc_degree(dst2, ones, zeros, half):
    base = half * _SPAN

    @functools.partial(
        pl.kernel,
        out_type=jax.ShapeDtypeStruct((2, _NS, _PSTRIPE, 128), jnp.float32),
        mesh=_sc_mesh(),
        compiler_params=pltpu.CompilerParams(use_tc_tiling_on_sc=False),
        scratch_types=[
            pltpu.VMEM_SHARED((_SROWS, 16), jnp.float32),
            pltpu.VMEM((16,), jnp.int32),
            pltpu.VMEM((16, 128), jnp.int32),
            pltpu.VMEM((128, 16), jnp.float32),
            pltpu.VMEM((_ZROWS, 16), jnp.float32),
            pltpu.VMEM((_PSTRIPE, 128), jnp.float32),
            pltpu.SemaphoreType.DMA,
        ],
    )
    def k(dst_hbm, ones_hbm, z_hbm, deg_hbm,
          acc_sh, irow, idst, ones_v, zbuf, packed, sem):
        c = lax.axis_index("c")
        s = lax.axis_index("s")
        _zero_acc(z_hbm, zbuf, acc_sh, s)
        pltpu.sync_copy(ones_hbm, ones_v)
        plsc.subcore_barrier()
        w = s * _NC + c
        tbase = w * 390 + jnp.minimum(w, 20)
        n_rows = 390 + jnp.where(w < 20, 1, 0)

        def do_chunk(row0, valid=None):
            irow[...] = jnp.minimum(row0 + lax.iota(jnp.int32, 16), _R - 1)
            pltpu.async_copy(dst_hbm.at[irow], idst, sem).wait()
            for j in range(16):
                def scatter_one(j=j):
                    _redirect(idst, j, base)
                    pltpu.sync_copy(ones_v, acc_sh.at[idst.at[j]], add=True)
                if valid is None:
                    scatter_one()
                else:
                    pl.when(j < valid)(scatter_one)

        def macro(m, carry):
            do_chunk(tbase + m * 16)
            return carry

        lax.fori_loop(0, 24, macro, 0)
        do_chunk(tbase + 384, n_rows - 384)
        plsc.subcore_barrier()
        _dump_packed(acc_sh, zbuf, packed, deg_hbm, c, s)

    return k(dst2, ones, zeros)


def _unpack128(a):
    cols = [a[:, k * 16:(k + 1) * 16] for k in range(8)]
    return jnp.reshape(jnp.stack(cols, axis=1), (_BLK, 16))


def _full(shape):
    return pl.BlockSpec(shape, lambda i: tuple(0 for _ in shape))


def _acc_specs():
    def span_map(t, ch):
        return lambda i: (ch, jnp.clip(i - t * _HBLK, 0, _HBLK - 1), 0)
    specs = []
    for t in range(_NCALL):
        specs.append(pl.BlockSpec((1, _PBLK, 128), span_map(t, 0)))
        specs.append(pl.BlockSpec((1, _PBLK, 128), span_map(t, 1)))
    return specs


def _sel_unpack(i, refs):
    acc_a = _unpack128(refs[0][0])
    acc_b = _unpack128(refs[1][0])
    for t in range(1, _NCALL):
        in_t = i >= t * _HBLK
        acc_a = jnp.where(in_t, _unpack128(refs[2 * t][0]), acc_a)
        acc_b = jnp.where(in_t, _unpack128(refs[2 * t + 1][0]), acc_b)
    return jnp.concatenate([acc_a, acc_b], axis=1)


def _tc_prologue(x2, degs, emb, w1, b1, w2, b2, wc0):
    def body(x_ref, dg0a, dg0b, dg1a, dg1b, dg2a, dg2b, emb_ref,
             w1r, b1r, w2r, b2r,
             wc0r, h0_ref, hta_ref, htb_ref, dis_ref):
        i = pl.program_id(0)
        t = jnp.maximum(jnp.dot(emb_ref[...], w1r[...],
                                preferred_element_type=jnp.float32) + b1r[...], 0.0)
        t = jnp.maximum(jnp.dot(t, w2r[...],
                                preferred_element_type=jnp.float32) + b2r[...], 0.0)
        x = x_ref[:, 0]
        oh = (x[:, None] == lax.broadcasted_iota(jnp.int32, (_BLK, _V), 1)
              ).astype(jnp.float32)
        h0 = jnp.dot(oh, t, preferred_element_type=jnp.float32)
        du = _sel_unpack(i, [dg0a, dg0b, dg1a, dg1b, dg2a, dg2b])
        deg = du[:, 0:1] + du[:, 16:17] + 1.0
        dis = lax.rsqrt(deg)
        h0_ref[...] = h0
        dis_ref[...] = dis
        ht = jnp.dot(h0, wc0r[...], preferred_element_type=jnp.float32) * dis
        ht3 = jnp.reshape(ht, (_PBLK, 8, 32))
        for k in range(8):
            hta_ref[:, k * 16:(k + 1) * 16] = ht3[:, k, :16]
            htb_ref[:, k * 16:(k + 1) * 16] = ht3[:, k, 16:]

    return pl.pallas_call(
        body,
        grid=(_NBLK,),
        in_specs=[pl.BlockSpec((_BLK, 1), lambda i: (i, 0))] + _acc_specs() + [
            _full((_V, _C)), _full((_C, _C)), _full((1, _C)),
            _full((_C, _C)), _full((1, _C)), _full((_C, _C)),
        ],
        out_specs=[
            pl.BlockSpec((_BLK, _C), lambda i: (i, 0)),
            pl.BlockSpec((_PBLK, 128), lambda i: (i, 0)),
            pl.BlockSpec((_PBLK, 128), lambda i: (i, 0)),
            pl.BlockSpec((_BLK, 1), lambda i: (i, 0)),
        ],
        out_shape=[
            jax.ShapeDtypeStruct((_N, _C), jnp.float32),
            jax.ShapeDtypeStruct((_PROWS, 128), jnp.float32),
            jax.ShapeDtypeStruct((_PROWS, 128), jnp.float32),
            jax.ShapeDtypeStruct((_N, 1), jnp.float32),
        ],
    )(x2, degs[0], degs[0], degs[1], degs[1], degs[2], degs[2],
      emb, w1, b1, w2, b2, wc0)


def _tc_conv(h, accs, dis, wl, bl, wn):
    last = wn is None

    def body(h_ref, a0, b0_, a1, b1_, a2, b2_, dis_ref, wlr, blr, *rest):
        if last:
            (h_out,) = rest
        else:
            wnr, h_out, hta_out, htb_out = rest
        i = pl.program_id(0)
        dis = dis_ref[...]
        hw = jnp.dot(h_ref[...], wlr[...], preferred_element_type=jnp.float32)
        acc = _sel_unpack(i, [a0, b0_, a1, b1_, a2, b2_])
        out = jnp.maximum(dis * acc + (dis * dis) * hw + blr[...], 0.0)
        h_out[...] = out
        if not last:
            ht = jnp.dot(out, wnr[...], preferred_element_type=jnp.float32) * dis
            ht3 = jnp.reshape(ht, (_PBLK, 8, 32))
            for k in range(8):
                hta_out[:, k * 16:(k + 1) * 16] = ht3[:, k, :16]
                htb_out[:, k * 16:(k + 1) * 16] = ht3[:, k, 16:]

    in_specs = [pl.BlockSpec((_BLK, _C), lambda i: (i, 0))] + _acc_specs() + [
        pl.BlockSpec((_BLK, 1), lambda i: (i, 0)),
        _full((_C, _C)), _full((1, _C)),
    ]
    out_specs = [pl.BlockSpec((_BLK, _C), lambda i: (i, 0))]
    out_shape = [jax.ShapeDtypeStruct((_N, _C), jnp.float32)]
    args = [h, accs[0], accs[0], accs[1], accs[1], accs[2], accs[2],
            dis, wl, bl]
    if not last:
        in_specs.append(_full((_C, _C)))
        out_specs.append(pl.BlockSpec((_PBLK, 128), lambda i: (i, 0)))
        out_specs.append(pl.BlockSpec((_PBLK, 128), lambda i: (i, 0)))
        out_shape.append(jax.ShapeDtypeStruct((_PROWS, 128), jnp.float32))
        out_shape.append(jax.ShapeDtypeStruct((_PROWS, 128), jnp.float32))
        args.append(wn)

    res = pl.pallas_call(
        body, grid=(_NBLK,), in_specs=in_specs,
        out_specs=out_specs, out_shape=out_shape,
    )(*args)
    return (res[0], None, None) if last else (res[0], res[1], res[2])


def _tc_pool(batch2, h):
    def body(b_ref, h_ref, g_ref):
        i = pl.program_id(0)

        @pl.when(i == 0)
        def _():
            g_ref[...] = jnp.zeros_like(g_ref)

        b = b_ref[:, 0]
        row = i * _BLK + lax.broadcasted_iota(jnp.int32, (_BLK, _G), 0)
        oh = ((b[:, None] == lax.broadcasted_iota(jnp.int32, (_BLK, _G), 1))
              & (row < _N)).astype(jnp.float32)
        g_ref[...] += lax.dot_general(
            oh, h_ref[...], (((0,), (0,)), ((), ())),
            preferred_element_type=jnp.float32)

    return pl.pallas_call(
        body,
        grid=(_NBLK,),
        in_specs=[
            pl.BlockSpec((_BLK, 1), lambda i: (i, 0)),
            pl.BlockSpec((_BLK, _C), lambda i: (i, 0)),
        ],
        out_specs=pl.BlockSpec((_G, _C), lambda i: (0, 0)),
        out_shape=jax.ShapeDtypeStruct((_G, _C), jnp.float32),
    )(batch2, h)


def _tc_head(g, w1, b1, w2, b2, wp, bp):
    def body(g_ref, w1r, b1r, w2r, b2r, wpr, bpr, o_ref):
        t = jnp.maximum(jnp.dot(g_ref[...], w1r[...],
                                preferred_element_type=jnp.float32) + b1r[...], 0.0)
        t = jnp.maximum(jnp.dot(t, w2r[...],
                                preferred_element_type=jnp.float32) + b2r[...], 0.0)
        o_ref[...] = jnp.dot(t, wpr[...],
                             preferred_element_type=jnp.float32) + bpr[...]

    f0 = lambda shape: pl.BlockSpec(shape, lambda: tuple(0 for _ in shape))
    return pl.pallas_call(
        body,
        in_specs=[f0((_G, _C)), f0((_C, _C)), f0((1, _C)),
                  f0((_C, _C)), f0((1, _C)), f0((_C, 1)),
                  f0((1, 1))],
        out_specs=f0((_G, 1)),
        out_shape=jax.ShapeDtypeStruct((_G, 1), jnp.float32),
    )(g, w1, b1, w2, b2, wp, bp)


def kernel(x, edge_index, batch, params):
    src2 = edge_index[0].astype(jnp.int32).reshape(_R, 128)
    dst2 = edge_index[1].astype(jnp.int32).reshape(_R, 128)
    zeros = jnp.zeros((_ZROWS, 16), jnp.float32)
    ones = jnp.ones((128, 16), jnp.float32)

    x2 = x.astype(jnp.int32).reshape(_N, 1)
    batch2 = batch.astype(jnp.int32).reshape(_N, 1)

    p = params
    b = lambda v: v.reshape(1, -1)

    degs = [_sc_degree(dst2, ones, zeros, t).reshape(2, _AROWS, 128)
            for t in range(_NCALL)]

    h, hta, htb, dis = _tc_prologue(
        x2, degs, p["emb"],
        p["pre"][0]["W"], b(p["pre"][0]["b"]),
        p["pre"][1]["W"], b(p["pre"][1]["b"]),
        p["convs"][0]["W"])

    n_convs = len(p["convs"])
    for l in range(n_convs):
        ha = hta.reshape(_NPAD, 16)
        hb = htb.reshape(_NPAD, 16)
        accs = [_sc_edge_scatter(src2, dst2, ha, hb, zeros, t
                                 ).reshape(2, _AROWS, 128)
                for t in range(_NCALL)]
        wn = p["convs"][l + 1]["W"] if l + 1 < n_convs else None
        h, hta, htb = _tc_conv(h, accs, dis, p["convs"][l]["W"],
                               b(p["convs"][l]["b"]), wn)

    g = _tc_pool(batch2, h)
    return _tc_head(
        g,
        p["post"][0]["W"], b(p["post"][0]["b"]),
        p["post"][1]["W"], b(p["post"][1]["b"]),
        p["prop"]["W"], b(p["prop"]["b"].reshape(1, 1)))
